# baseline (speedup 1.0000x reference)
import jax
import jax.numpy as jnp
from jax import lax
from jax.experimental import pallas as pl
from jax.experimental.pallas import tpu as pltpu
from jax.experimental.pallas import tpu_sc as plsc

_B, _L, _E = 16384, 50, 64
_S = 20
_SP = 32
_LE = _L * _E
_BB = 512

_INFO = plsc.get_sparse_core_info()
_NC, _NS, _LANES = _INFO.num_cores, _INFO.num_subcores, _INFO.num_lanes
_NW = _NC * _NS
_BPW = _B // _NW
_GW = 128
_CHUNK = 128
_NCHUNK = _BPW // _CHUNK


def _sc_gather(table_hbm, idx_hbm, sb_hbm, idx_v, rows_v, sem):
    wid = lax.axis_index("s") * _NC + lax.axis_index("c")
    pltpu.sync_copy(idx_hbm.at[pl.ds(wid * _NCHUNK, _NCHUNK)], idx_v)
    for c in range(_NCHUNK):
        pltpu.async_copy(
            table_hbm.at[idx_v.at[c]],
            rows_v.at[pl.ds(c * _CHUNK, _CHUNK)],
            sem,
        ).wait()
    pltpu.sync_copy(rows_v, sb_hbm.at[pl.ds(wid * _BPW, _BPW)])


def _tc_body(sb_ref, pos_ref, item_ref, embed_ref, out_ref):
    sb = sb_ref[...][:, :1]
    out_ref[...] = embed_ref[...] + sb + (pos_ref[...] + item_ref[...])


def kernel(session_embed, session_index, session_bias, position_bias, item_bias):
    embed2d = session_embed.reshape(_B, _LE)
    idx1d = session_index.astype(jnp.int32).reshape(_B)
    table16 = jnp.broadcast_to(
        jnp.pad(session_bias.reshape(_S), (0, _SP - _S)).reshape(_SP, 1),
        (_SP, _GW),
    )
    pos2d = jnp.broadcast_to(position_bias, (1, _L, _E)).reshape(1, _LE)
    item2d = jnp.broadcast_to(item_bias, (1, _L, _E)).reshape(1, _LE)

    mesh = plsc.VectorSubcoreMesh(core_axis_name="c", subcore_axis_name="s")
    sb16 = pl.kernel(
        _sc_gather,
        mesh=mesh,
        out_type=jax.ShapeDtypeStruct((_B, _GW), jnp.float32),
        scratch_types=[
            pltpu.VMEM((_NCHUNK, _CHUNK), jnp.int32),
            pltpu.VMEM((_BPW, _GW), jnp.float32),
            pltpu.SemaphoreType.DMA,
        ],
    )(table16, idx1d.reshape(_B // _CHUNK, _CHUNK))

    out2d = pl.pallas_call(
        _tc_body,
        grid=(_B // _BB,),
        in_specs=[
            pl.BlockSpec((_BB, _GW), lambda i: (i, 0)),
            pl.BlockSpec((1, _LE), lambda i: (0, 0)),
            pl.BlockSpec((1, _LE), lambda i: (0, 0)),
            pl.BlockSpec((_BB, _LE), lambda i: (i, 0)),
        ],
        out_specs=pl.BlockSpec((_BB, _LE), lambda i: (i, 0)),
        out_shape=jax.ShapeDtypeStruct((_B, _LE), jnp.float32),
        compiler_params=pltpu.CompilerParams(
            dimension_semantics=("arbitrary",),
        ),
    )(sb16, pos2d, item2d, embed2d)
    return out2d.reshape(_B, _L, _E)
